# baseline (speedup 1.0000x reference)
import jax
import jax.numpy as jnp
from jax import lax
from jax.experimental import pallas as pl
from jax.experimental.pallas import tpu as pltpu, tpu_sc as plsc

NCB = 8
V = 100000
D = 64
N = 1024 * 50
NW = 32
PER_W = N * NCB // NW
CHUNK = 128
NCHUNK = PER_W // CHUNK
POS_PER_CHUNK = CHUNK // NCB
NPAIR = NCHUNK // 2 - 1


def _body(idx_hbm, w_hbm, out_hbm, idx_buf, rows_buf, ob,
          gsem0, gsem1, osem0, osem1):
    nc = 2
    wid = lax.axis_index("s") * nc + lax.axis_index("c")
    out_base = wid * (PER_W // NCB)

    pltpu.sync_copy(idx_hbm.at[wid], idx_buf)
    offs = (jnp.arange(16, dtype=jnp.int32) % NCB) * V

    def add_offs(k, _):
        for j in range(CHUNK // 16):
            sl = pl.ds(j * 16, 16)
            idx_buf[k, sl] = idx_buf[k, sl] + offs
        return 0

    lax.fori_loop(0, NCHUNK, add_offs, 0)

    gsems = (gsem0, gsem1)
    osems = (osem0, osem1)
    w0 = w_hbm.at[0]

    def issue(k, b):
        pltpu.async_copy(w0.at[idx_buf.at[k]], rows_buf.at[b], gsems[b])

    def drain_gather(b):
        pltpu.make_async_copy(
            w0.at[idx_buf.at[0]], rows_buf.at[b], gsems[b]
        ).wait()

    def accumulate(b):
        def pos_body(p, _):
            r0 = NCB * p
            for j in range(D // 16):
                sl = pl.ds(j * 16, 16)
                acc = rows_buf[b, r0, sl]
                for c in range(1, NCB):
                    acc = acc + rows_buf[b, r0 + c, sl]
                ob[b, p, sl] = acc
            return 0

        lax.fori_loop(0, POS_PER_CHUNK, pos_body, 0)

    def flush(k, b):
        pltpu.async_copy(
            ob.at[b],
            out_hbm.at[pl.ds(out_base + k * POS_PER_CHUNK, POS_PER_CHUNK)],
            osems[b],
        )

    def drain_flush(b):
        pltpu.make_async_copy(
            out_hbm.at[pl.ds(0, POS_PER_CHUNK)], ob.at[b], osems[b]
        ).wait()

    issue(0, 0)
    issue(1, 1)

    def pair_body(i, _):
        k = 2 * i
        drain_gather(0)

        @pl.when(i > 0)
        def _():
            drain_flush(0)

        accumulate(0)
        flush(k, 0)
        issue(k + 2, 0)
        drain_gather(1)

        @pl.when(i > 0)
        def _():
            drain_flush(1)

        accumulate(1)
        flush(k + 1, 1)
        issue(k + 3, 1)
        return 0

    lax.fori_loop(0, NPAIR, pair_body, 0)

    drain_gather(0)
    drain_flush(0)
    accumulate(0)
    flush(NCHUNK - 2, 0)
    drain_gather(1)
    drain_flush(1)
    accumulate(1)
    flush(NCHUNK - 1, 1)
    drain_flush(0)
    drain_flush(1)


def _run(idx3, weight):
    f = pl.kernel(
        _body,
        out_type=jax.ShapeDtypeStruct((N, D), jnp.float32),
        mesh=plsc.VectorSubcoreMesh(core_axis_name="c", subcore_axis_name="s"),
        scratch_types=[
            pltpu.VMEM((NCHUNK, CHUNK), jnp.int32),
            pltpu.VMEM((2, CHUNK, D), jnp.float32),
            pltpu.VMEM((2, POS_PER_CHUNK, D), jnp.float32),
            pltpu.SemaphoreType.DMA,
            pltpu.SemaphoreType.DMA,
            pltpu.SemaphoreType.DMA,
            pltpu.SemaphoreType.DMA,
        ],
        compiler_params=pltpu.CompilerParams(use_tc_tiling_on_sc=False),
    )
    return f(idx3, weight)


@jax.jit
def kernel(indices, weight):
    B, S, C = indices.shape
    idx3 = indices.astype(jnp.int32).reshape(NW, NCHUNK, CHUNK)
    out = _run(idx3, weight)
    return out.reshape(B, S, D)
